# baseline (speedup 1.0000x reference)
import jax
import jax.numpy as jnp
from jax.experimental import pallas as pl
from jax.experimental.pallas import tpu as pltpu
from jax.experimental.pallas import tpu_sc as plsc

_NUM_CORES = 2
_NUM_SUBCORES = 16
_WINDOW = 128


def kernel(t, pos_embedding):
    (batch,) = t.shape
    vocab, dim = pos_embedding.shape
    num_steps = batch // _WINDOW

    mesh = plsc.VectorSubcoreMesh(core_axis_name="c", subcore_axis_name="s")

    @pl.kernel(
        mesh=mesh,
        out_type=jax.ShapeDtypeStruct((batch, dim), pos_embedding.dtype),
    )
    def gather_kernel(table_hbm, idx_hbm, out_hbm):
        def body(idx_v, rows_v):
            pltpu.sync_copy(table_hbm.at[idx_v.at[0]], rows_v)

        pltpu.emit_pipeline(
            body,
            grid=(num_steps,),
            in_specs=[pl.BlockSpec((1, _WINDOW), index_map=lambda i: (0, i))],
            out_specs=[pl.BlockSpec((_WINDOW, dim), index_map=lambda i: (i, 0))],
            core_axis_name=("c", "s"),
            dimension_semantics=(pltpu.PARALLEL,),
        )(idx_hbm, out_hbm)

    return gather_kernel(pos_embedding, t.astype(jnp.int32).reshape(1, batch))


# speedup vs baseline: 1.0323x; 1.0323x over previous
import functools

import jax
import jax.numpy as jnp
from jax import lax
from jax.experimental import pallas as pl
from jax.experimental.pallas import tpu as pltpu
from jax.experimental.pallas import tpu_sc as plsc

_NUM_CORES = 2
_NUM_SUBCORES = 16
_NUM_WORKERS = _NUM_CORES * _NUM_SUBCORES
_NUM_CHUNKS = 4


def kernel(t, pos_embedding):
    (batch,) = t.shape
    vocab, dim = pos_embedding.shape
    b_per_w = batch // _NUM_WORKERS
    chunk = b_per_w // _NUM_CHUNKS
    assert chunk % 8 == 0

    mesh = plsc.VectorSubcoreMesh(core_axis_name="c", subcore_axis_name="s")

    @functools.partial(
        pl.kernel,
        mesh=mesh,
        out_type=jax.ShapeDtypeStruct((batch, dim), pos_embedding.dtype),
        scratch_types=[
            pltpu.VMEM((b_per_w,), jnp.int32),
            pltpu.VMEM((b_per_w, dim), jnp.float32),
            pltpu.SemaphoreType.DMA,
            pltpu.SemaphoreType.DMA,
        ],
    )
    def gather_kernel(table_hbm, idx_hbm, out_hbm, idx_v, rows_v, gsem, wsem):
        wid = lax.axis_index("s") * _NUM_CORES + lax.axis_index("c")
        base = wid * b_per_w
        pltpu.sync_copy(idx_hbm.at[pl.ds(base, b_per_w)], idx_v)
        gathers = []
        for k in range(_NUM_CHUNKS):
            gathers.append(pltpu.async_copy(
                table_hbm.at[idx_v.at[pl.ds(k * chunk, chunk)]],
                rows_v.at[pl.ds(k * chunk, chunk)],
                gsem,
            ))
        writes = []
        for k in range(_NUM_CHUNKS):
            gathers[k].wait()
            writes.append(pltpu.async_copy(
                rows_v.at[pl.ds(k * chunk, chunk)],
                out_hbm.at[pl.ds(base + k * chunk, chunk)],
                wsem,
            ))
        for w in writes:
            w.wait()

    return gather_kernel(pos_embedding, t.astype(jnp.int32))
